# Initial kernel scaffold; baseline (speedup 1.0000x reference)
import jax
import jax.numpy as jnp
from jax.experimental import pallas as pl


def kernel(x_in, codebook):
    raise NotImplementedError("write your pallas kernel here")


# speedup vs baseline: 2.6171x; 2.6171x over previous
import jax
import jax.numpy as jnp
from jax.experimental import pallas as pl
from jax.experimental.pallas import tpu as pltpu

_B, _D, _N = 32, 64, 576
_K = 1024


def _vq_kernel(x_ref, emb_ref, out_ref, idx_ref):
    x = x_ref[0]
    emb = emb_ref[...]
    scores = jax.lax.dot_general(
        emb, x, (((1,), (0,)), ((), ())),
        preferred_element_type=jnp.float32)
    x2 = jnp.sum(x * x, axis=0)
    e2 = jnp.sum(emb * emb, axis=1)
    d2 = x2[None, :] + e2[:, None] - 2.0 * scores
    d2 = jnp.maximum(d2, 0.0)
    idx = jnp.argmin(d2, axis=0)
    idx_ref[0, :, 0] = idx
    onehot = (jax.lax.broadcasted_iota(jnp.int32, (_K, _N), 0)
              == idx[None, :]).astype(jnp.float32)
    out = jax.lax.dot_general(
        emb, onehot, (((0,), (0,)), ((), ())),
        preferred_element_type=jnp.float32)
    out_ref[0] = out


def kernel(x_in, codebook):
    out, idx = pl.pallas_call(
        _vq_kernel,
        grid=(_B,),
        in_specs=[
            pl.BlockSpec((1, _D, _N), lambda b: (b, 0, 0)),
            pl.BlockSpec((_K, _D), lambda b: (0, 0)),
        ],
        out_specs=[
            pl.BlockSpec((1, _D, _N), lambda b: (b, 0, 0)),
            pl.BlockSpec((1, _N, 1), lambda b: (b, 0, 0)),
        ],
        out_shape=[
            jax.ShapeDtypeStruct((_B, _D, _N), jnp.float32),
            jax.ShapeDtypeStruct((_B, _N, 1), jnp.int32),
        ],
        compiler_params=pltpu.CompilerParams(
            dimension_semantics=("parallel",)),
    )(x_in, codebook)
    return out, idx


# speedup vs baseline: 3.4416x; 1.3150x over previous
import jax
import jax.numpy as jnp
from jax.experimental import pallas as pl
from jax.experimental.pallas import tpu as pltpu

_B, _D, _N = 32, 64, 576
_K = 1024


def _vq_kernel(x_ref, emb_ref, out_ref, idx_ref):
    x = x_ref[0]
    emb = emb_ref[...]
    scores = jax.lax.dot_general(
        emb, x, (((1,), (0,)), ((), ())),
        preferred_element_type=jnp.float32)
    e2 = jnp.sum(emb * emb, axis=1)
    d2 = e2[:, None] - 2.0 * scores
    idx = jnp.argmin(d2, axis=0)
    idx_ref[0, 0, :] = idx
    onehot = (jax.lax.broadcasted_iota(jnp.int32, (_K, _N), 0)
              == idx[None, :]).astype(jnp.float32)
    out = jax.lax.dot_general(
        emb, onehot, (((0,), (0,)), ((), ())),
        preferred_element_type=jnp.float32)
    out_ref[0] = out


def kernel(x_in, codebook):
    out, idx = pl.pallas_call(
        _vq_kernel,
        grid=(_B,),
        in_specs=[
            pl.BlockSpec((1, _D, _N), lambda b: (b, 0, 0)),
            pl.BlockSpec((_K, _D), lambda b: (0, 0)),
        ],
        out_specs=[
            pl.BlockSpec((1, _D, _N), lambda b: (b, 0, 0)),
            pl.BlockSpec((1, 1, _N), lambda b: (b, 0, 0)),
        ],
        out_shape=[
            jax.ShapeDtypeStruct((_B, _D, _N), jnp.float32),
            jax.ShapeDtypeStruct((_B, 1, _N), jnp.int32),
        ],
        compiler_params=pltpu.CompilerParams(
            dimension_semantics=("parallel",)),
    )(x_in, codebook)
    return out, jnp.reshape(idx, (_B, _N, 1))
